# baseline (speedup 1.0000x reference)
import functools

import jax
import jax.numpy as jnp
from jax import lax
from jax.experimental import pallas as pl
from jax.experimental.pallas import tpu as pltpu
from jax.experimental.pallas import tpu_sc as plsc

N = 10000
E = 320000
D = 128
H = 16
NC, NS = 2, 16
NT = NC * NS
SCH = 512
KB = 20
EPAD = KB * SCH
RING = 8
LOOK = 3
EPT = E // NT
PADT = EPAD - EPT
NPAD = 10112
RPW = NPAD // NS
PK = 128 // H
NR = N * H // 128
NRP = NPAD * H // 128
NTRASH = NPAD - N


def _xw_body(x_ref, b_ref, o_ref):
    x3 = x_ref[...].reshape(NR, PK, D)
    acc = jnp.dot(x3[:, 0, :], b_ref[0],
                  preferred_element_type=jnp.float32)
    for a in range(1, PK):
        acc += jnp.dot(x3[:, a, :], b_ref[a],
                       preferred_element_type=jnp.float32)
    o_ref[...] = acc


_sc_mesh = plsc.VectorSubcoreMesh(core_axis_name="c", subcore_axis_name="s")


@functools.partial(
    pl.kernel,
    out_type=[
        jax.ShapeDtypeStruct((NC, NPAD, H), jnp.float32),
        jax.ShapeDtypeStruct((NC, NPAD), jnp.float32),
    ],
    mesh=_sc_mesh,
    scratch_types=[
        pltpu.VMEM((EPAD,), jnp.int32),
        pltpu.VMEM((EPAD,), jnp.int32),
        pltpu.VMEM((RING, SCH, H), jnp.float32),
        pltpu.VMEM((SCH,), jnp.float32),
        pltpu.VMEM_SHARED((NPAD, H), jnp.float32),
        pltpu.VMEM_SHARED((NPAD,), jnp.float32),
        pltpu.SemaphoreType.DMA((RING,)),
        pltpu.SemaphoreType.DMA((RING,)),
        pltpu.SemaphoreType.DMA,
    ],
    compiler_params=pltpu.CompilerParams(use_tc_tiling_on_sc=False),
)
def _edge_scatter(xw_hbm, ei_hbm, psrc_hbm, pdst_hbm, zrow_hbm, zcnt_hbm,
                  agg_out, cnt_out,
                  src_v, dst_v, rows_v, ones_v,
                  agg_sh, cnt_sh, gsem, ssem, csem):
    c = lax.axis_index("c")
    s = lax.axis_index("s")
    t = c * NS + s
    lastw = N - (NS - 1) * RPW

    @pl.when(jnp.logical_and(c == 0, s < NS - 1))
    def _():
        pltpu.sync_copy(xw_hbm.at[pl.ds(s * RPW, RPW)],
                        agg_sh.at[pl.ds(s * RPW, RPW)])

    @pl.when(jnp.logical_and(c == 0, s == NS - 1))
    def _():
        pltpu.sync_copy(xw_hbm.at[pl.ds((NS - 1) * RPW, lastw)],
                        agg_sh.at[pl.ds((NS - 1) * RPW, lastw)])
        pltpu.sync_copy(zrow_hbm.at[pl.ds(0, NTRASH)],
                        agg_sh.at[pl.ds(N, NTRASH)])

    @pl.when(c == 1)
    def _():
        pltpu.sync_copy(zrow_hbm.at[pl.ds(s * RPW, RPW)],
                        agg_sh.at[pl.ds(s * RPW, RPW)])

    pltpu.sync_copy(zcnt_hbm.at[pl.ds(s * RPW, RPW)],
                    cnt_sh.at[pl.ds(s * RPW, RPW)])
    for k in range(SCH // 16):
        ones_v[pl.ds(k * 16, 16)] = jnp.ones((16,), jnp.float32)
    plsc.subcore_barrier()
    pltpu.sync_copy(ei_hbm.at[0, pl.ds(t * EPT, EPT)],
                    src_v.at[pl.ds(0, EPT)])
    pltpu.sync_copy(ei_hbm.at[1, pl.ds(t * EPT, EPT)],
                    dst_v.at[pl.ds(0, EPT)])
    pltpu.sync_copy(psrc_hbm.at[pl.ds(t * PADT, PADT)],
                    src_v.at[pl.ds(EPT, PADT)])
    pltpu.sync_copy(pdst_hbm.at[pl.ds(t * PADT, PADT)],
                    dst_v.at[pl.ds(EPT, PADT)])

    def start_gather(g, b):
        pltpu.async_copy(xw_hbm.at[src_v.at[pl.ds(g * SCH, SCH)]],
                         rows_v.at[b], gsem.at[b])

    def wait_gather(b):
        pltpu.make_async_copy(xw_hbm.at[src_v.at[pl.ds(0, SCH)]],
                              rows_v.at[b], gsem.at[b]).wait()

    def start_scatters(g, b):
        pltpu.async_copy(rows_v.at[b], agg_sh.at[dst_v.at[pl.ds(g * SCH, SCH)]],
                         ssem.at[b], add=True)
        pltpu.async_copy(ones_v, cnt_sh.at[dst_v.at[pl.ds(g * SCH, SCH)]],
                         csem, add=True)

    def wait_scatter(b):
        pltpu.make_async_copy(rows_v.at[b], agg_sh.at[dst_v.at[pl.ds(0, SCH)]],
                              ssem.at[b]).wait()

    for g in range(LOOK):
        start_gather(g, g)
    for g in range(KB):
        b = g % RING
        wait_gather(b)
        start_scatters(g, b)
        nxt = g + LOOK
        if nxt < KB:
            bn = nxt % RING
            if nxt >= RING:
                wait_scatter(bn)
            start_gather(nxt, bn)
    for g in range(KB - RING, KB):
        wait_scatter(g % RING)
    for g in range(KB):
        pltpu.make_async_copy(ones_v, cnt_sh.at[dst_v.at[pl.ds(0, SCH)]],
                              csem).wait()
    plsc.subcore_barrier()
    pltpu.sync_copy(agg_sh.at[pl.ds(s * RPW, RPW)],
                    agg_out.at[c, pl.ds(s * RPW, RPW)])
    pltpu.sync_copy(cnt_sh.at[pl.ds(s * RPW, RPW)],
                    cnt_out.at[c, pl.ds(s * RPW, RPW)])


def _mlp_body(p_ref, c_ref, g_ref, bias_ref, w1_ref, b1_ref, w2_ref, b2_ref,
              o_ref):
    s = p_ref[0, :NR, :] + p_ref[1, :NR, :]
    csum = c_ref[0] + c_ref[1]
    blocks = [jnp.dot(csum, g_ref[i], preferred_element_type=jnp.float32)
              for i in range(16)]
    cnt = jnp.stack(blocks, axis=1).reshape(NRP, 128)[:NR, :] + 1.0
    h = jnp.maximum(s / cnt + bias_ref[...], 0.0)
    h = jnp.maximum(
        jnp.dot(h, w1_ref[...], preferred_element_type=jnp.float32)
        + b1_ref[...], 0.0)
    y = (jnp.dot(h, w2_ref[...], preferred_element_type=jnp.float32)
         + b2_ref[...])
    o_ref[...] = jax.nn.sigmoid(y)


def _blockdiag(m, k):
    r, ccol = m.shape
    out = jnp.zeros((k, r, k, ccol), m.dtype)
    out = out.at[jnp.arange(k), :, jnp.arange(k), :].set(m)
    return out.reshape(k * r, k * ccol)


def kernel(x, edge_index, W, u, c, bias, W1, b1, W2, b2):
    del u, c
    ei = edge_index.astype(jnp.int32)
    psrc = jnp.zeros((NT * PADT,), jnp.int32)
    pdst = N + (jnp.arange(NT * PADT, dtype=jnp.int32) % NTRASH)

    b4 = jnp.stack([jnp.pad(W, ((0, 0), (a * H, 128 - (a + 1) * H)))
                    for a in range(PK)])
    xw8 = pl.pallas_call(
        _xw_body,
        out_shape=jax.ShapeDtypeStruct((NR, 128), jnp.float32),
    )(x, b4)

    zrow = jnp.zeros((NPAD, H), jnp.float32)
    zcnt = jnp.zeros((NPAD,), jnp.float32)
    parts, cnts = _edge_scatter(xw8.reshape(N, H), ei, psrc, pdst, zrow, zcnt)

    ii = jnp.arange(16).reshape(16, 1, 1)
    aa = jnp.arange(PK).reshape(1, PK, 1)
    hh = jnp.arange(H).reshape(1, 1, H)
    g = jnp.zeros((16, 128, 128), jnp.float32).at[
        jnp.broadcast_to(ii, (16, PK, H)),
        jnp.broadcast_to(8 * ii + aa, (16, PK, H)),
        jnp.broadcast_to(aa * H + hh, (16, PK, H))].set(1.0)

    y8 = pl.pallas_call(
        _mlp_body,
        out_shape=jax.ShapeDtypeStruct((NR, PK), jnp.float32),
    )(parts.reshape(NC, NRP, 128), cnts.reshape(NC, NPAD // 128, 128), g,
      jnp.tile(bias, PK).reshape(1, PK * H), _blockdiag(W1, PK),
      jnp.tile(b1, PK).reshape(1, PK * 8), _blockdiag(W2, PK),
      b2.reshape(1, 1))
    return y8.reshape(N, 1)
